# baseline (speedup 1.0000x reference)
import jax
import jax.numpy as jnp
from jax import lax
from jax.experimental import pallas as pl
from jax.experimental.pallas import tpu as pltpu
from jax.experimental.pallas import tpu_sc as plsc

B = 4096
L = 200
D = 64
NW = 32
BC = B // NW
LANES = 16
HALF_L = L // 2
GP = 130


def _body(xT_hbm, tok_hbm, posx_hbm, outT_hbm, x_v, j_v, g_v, t_v, ps_v,
          sg0, sg1, sw0, sw1):
    wid = lax.axis_index("s") * 2 + lax.axis_index("c")
    semg = (sg0, sg1)
    semw = (sw0, sw1)
    pltpu.sync_copy(xT_hbm.at[:, pl.ds(wid * BC, BC)], x_v)

    def fire(l, p):
        for cc in range(BC // LANES):
            xv = x_v[l, pl.ds(cc * LANES, LANES)]
            j_v[p, pl.ds(cc * LANES, LANES)] = lax.shift_right_logical(xv, 1)
        pltpu.async_copy(tok_hbm.at[j_v.at[p]],
                         g_v.at[p, :, pl.ds(0, 2 * D)], semg[p])
        pltpu.async_copy(posx_hbm.at[pl.ds(l * 8, 8)], ps_v.at[p], semg[p])

    def drain_g(p):
        pltpu.make_async_copy(tok_hbm.at[pl.ds(0, BC)],
                              g_v.at[p, :, pl.ds(0, 2 * D)], semg[p]).wait()
        pltpu.make_async_copy(posx_hbm.at[pl.ds(0, 8)], ps_v.at[p],
                              semg[p]).wait()

    def drain_w(p):
        pltpu.make_async_copy(t_v.at[p], outT_hbm.at[0, :, pl.ds(0, BC)],
                              semw[p]).wait()

    iota = lax.iota(jnp.int32, LANES)
    rows16 = [iota + cc * LANES for cc in range(BC // LANES)]

    def process(l, p):
        hcols = [(x_v[l, pl.ds(cc * LANES, LANES)] & 1) * D
                 for cc in range(BC // LANES)]

        @plsc.parallel_loop(0, D, unroll=8)
        def _(d):
            pv = ps_v[p, d // 8, pl.ds((d % 8) * LANES, LANES)]
            for cc in range(BC // LANES):
                gv = plsc.load_gather(g_v.at[p], [rows16[cc], hcols[cc] + d])
                t_v[p, d, pl.ds(cc * LANES, LANES)] = gv + pv

        pltpu.async_copy(t_v.at[p], outT_hbm.at[l, :, pl.ds(wid * BC, BC)],
                         semw[p])

    fire(0, 0)

    def step(t, carry):
        @pl.when(t > 0)
        def _():
            drain_w(1)

        fire(2 * t + 1, 1)
        drain_g(0)
        process(2 * t, 0)

        @pl.when(t < HALF_L - 1)
        def _():
            drain_w(0)
            fire(2 * t + 2, 0)

        drain_g(1)
        process(2 * t + 1, 1)
        return carry

    lax.fori_loop(0, HALF_L, step, 0)
    drain_w(0)
    drain_w(1)


@jax.jit
def _embed(xT, tok128, posx):
    mesh = plsc.VectorSubcoreMesh(core_axis_name="c", subcore_axis_name="s")
    kfn = pl.kernel(
        _body,
        out_type=jax.ShapeDtypeStruct((L, D, B), jnp.float32),
        mesh=mesh,
        scratch_types=[
            pltpu.VMEM((L, BC), jnp.int32),
            pltpu.VMEM((2, BC), jnp.int32),
            pltpu.VMEM((2, BC, GP), jnp.float32),
            pltpu.VMEM((2, D, BC), jnp.float32),
            pltpu.VMEM((2, 8, 128), jnp.float32),
            pltpu.SemaphoreType.DMA,
            pltpu.SemaphoreType.DMA,
            pltpu.SemaphoreType.DMA,
            pltpu.SemaphoreType.DMA,
        ],
        compiler_params=pltpu.CompilerParams(use_tc_tiling_on_sc=True,
                                             needs_layout_passes=False),
    )
    return kfn(xT, tok128, posx)


def kernel(x, token_table, pos_table):
    xT = x.T.astype(jnp.int32)
    tok128 = token_table.reshape(500000, 128)
    posx = jnp.broadcast_to(pos_table.reshape(L * D, 1),
                            (L * D, LANES)).reshape(L * 8, 128)
    outT = _embed(xT, tok128, posx)
    return jnp.transpose(outT, (2, 0, 1))


# speedup vs baseline: 1.0883x; 1.0883x over previous
import jax
import jax.numpy as jnp
from jax import lax
from jax.experimental import pallas as pl
from jax.experimental.pallas import tpu as pltpu
from jax.experimental.pallas import tpu_sc as plsc

B = 4096
L = 200
D = 64
N = B * L
NW = 32
CH = 100
ROWS_PER_W = N // NW
NCH = ROWS_PER_W // CH
K = 4
NG = NCH // K
NGH = NG // 2
LANES = 16
VECS_PER_ROW = D // LANES


def _body(x_hbm, tok_hbm, pos_hbm, out_hbm, idx_v, rows_v, pos_v,
          sg0, sg1, sw0, sw1):
    wid = lax.axis_index("s") * 2 + lax.axis_index("c")
    semg = (sg0, sg1)
    semw = (sw0, sw1)
    pltpu.sync_copy(x_hbm.at[pl.ds(wid * NCH, NCH)], idx_v)
    pltpu.sync_copy(pos_hbm, pos_v)

    def fire_gathers(g, p):
        for b in range(K):
            c = g * K + b
            pltpu.async_copy(tok_hbm.at[idx_v.at[c]], rows_v.at[p, b], semg[p])

    def drain_gathers(p):
        for b in range(K):
            pltpu.make_async_copy(
                tok_hbm.at[pl.ds(0, CH)], rows_v.at[p, b], semg[p]).wait()

    def add_pos(p, b):
        pbase = (b % 2) * (CH * D)

        def row(r, carry):
            for d in range(VECS_PER_ROW):
                pv = pos_v[pl.ds(pbase + r * D + d * LANES, LANES)]
                plsc.addupdate(rows_v.at[p, b, r, pl.ds(d * LANES, LANES)], pv)
            return carry

        lax.fori_loop(0, CH, row, 0, unroll=4)

    def process_group(g, p):
        base = wid * NCH + g * K
        for b in range(K):
            add_pos(p, b)
            pltpu.async_copy(rows_v.at[p, b], out_hbm.at[base + b], semw[p])

    def drain_writes(p):
        for b in range(K):
            pltpu.make_async_copy(
                rows_v.at[p, b], out_hbm.at[0], semw[p]).wait()

    fire_gathers(0, 0)

    def step(t, carry):
        @pl.when(t > 0)
        def _():
            drain_writes(1)

        fire_gathers(2 * t + 1, 1)
        drain_gathers(0)
        process_group(2 * t, 0)

        @pl.when(t < NGH - 1)
        def _():
            drain_writes(0)
            fire_gathers(2 * t + 2, 0)

        drain_gathers(1)
        process_group(2 * t + 1, 1)
        return carry

    lax.fori_loop(0, NGH, step, 0)
    drain_writes(0)
    drain_writes(1)


@jax.jit
def _embed(x2d, token_table, pos_flat):
    mesh = plsc.VectorSubcoreMesh(core_axis_name="c", subcore_axis_name="s")
    kfn = pl.kernel(
        _body,
        out_type=jax.ShapeDtypeStruct((N // CH, CH, D), jnp.float32),
        mesh=mesh,
        scratch_types=[
            pltpu.VMEM((NCH, CH), jnp.int32),
            pltpu.VMEM((2, K, CH, D), jnp.float32),
            pltpu.VMEM((L * D,), jnp.float32),
            pltpu.SemaphoreType.DMA,
            pltpu.SemaphoreType.DMA,
            pltpu.SemaphoreType.DMA,
            pltpu.SemaphoreType.DMA,
        ],
        compiler_params=pltpu.CompilerParams(use_tc_tiling_on_sc=False),
    )
    return kfn(x2d, token_table, pos_flat)


def kernel(x, token_table, pos_table):
    x2d = x.reshape(N // CH, CH).astype(jnp.int32)
    pos_flat = pos_table.reshape(L * D)
    out = _embed(x2d, token_table, pos_flat)
    return out.reshape(B, L, D)
